# baseline (speedup 1.0000x reference)
import functools

import jax
import jax.numpy as jnp
from jax import lax
from jax.experimental import pallas as pl
from jax.experimental.pallas import tpu as pltpu
from jax.experimental.pallas import tpu_sc as plsc

EPS = 1e-5
NBLK = 512
MB = 512
SC_CP = 128


def _three_nn_body(S, xyz1t_ref, xyz2_ref, idx_ref, w_ref):
    b = pl.program_id(0)
    x1t = xyz1t_ref[0]
    nb = x1t.shape[1]
    x1r = [x1t[c:c + 1, :] for c in range(3)]
    BIGF = jnp.float32(1e30)
    base_iota = lax.broadcasted_iota(jnp.int32, (8, nb), 0)

    def row(r, carry):
        m1, m2, m3, i1, i2, i3 = carry
        x2c = xyz2_ref[0, pl.ds(r * 8, 8), :]
        d = None
        for c in range(3):
            diff = x2c[:, c:c + 1] - x1r[c]
            d = diff * diff if d is None else d + diff * diff
        s = base_iota + r * 8
        c1 = d < m1
        c2 = d < m2
        c3 = d < m3
        m3n = jnp.where(c3, jnp.where(c2, m2, d), m3)
        i3n = jnp.where(c3, jnp.where(c2, i2, s), i3)
        m2n = jnp.where(c2, jnp.where(c1, m1, d), m2)
        i2n = jnp.where(c2, jnp.where(c1, i1, s), i2)
        m1n = jnp.where(c1, d, m1)
        i1n = jnp.where(c1, s, i1)
        return (m1n, m2n, m3n, i1n, i2n, i3n)

    init = (jnp.full((8, nb), BIGF), jnp.full((8, nb), BIGF),
            jnp.full((8, nb), BIGF), jnp.zeros((8, nb), jnp.int32),
            jnp.zeros((8, nb), jnp.int32), jnp.zeros((8, nb), jnp.int32))
    m1, m2, m3, i1, i2, i3 = lax.fori_loop(0, S // 8, row, init, unroll=4)
    mm = jnp.concatenate([m1, m2, m3], axis=0)
    ii = jnp.concatenate([i1, i2, i3], axis=0)
    idxs, vals = [], []
    for k in range(3):
        m = jnp.min(mm, axis=0, keepdims=True)
        i = jnp.min(jnp.where(mm == m, ii, S), axis=0, keepdims=True)
        vals.append(m)
        idxs.append(i)
        if k < 2:
            mm = jnp.where((mm == m) & (ii == i), BIGF, mm)
    recip = [1.0 / (v + 1e-8) for v in vals]
    norm = recip[0] + recip[1] + recip[2]
    idx_ref[0] = jnp.concatenate(idxs, axis=0) + b * S
    w_ref[0] = jnp.concatenate([r / norm for r in recip], axis=0)


def _three_nn(xyz1t, xyz2):
    B, _, N = xyz1t.shape
    S = xyz2.shape[1]
    grid = (B, N // NBLK)
    return pl.pallas_call(
        functools.partial(_three_nn_body, S),
        grid=grid,
        in_specs=[
            pl.BlockSpec((1, 3, NBLK), lambda b, i: (b, 0, i)),
            pl.BlockSpec((1, S, 3), lambda b, i: (b, 0, 0)),
        ],
        out_specs=[
            pl.BlockSpec((1, 3, NBLK), lambda b, i: (b, 0, i)),
            pl.BlockSpec((1, 3, NBLK), lambda b, i: (b, 0, i)),
        ],
        out_shape=[
            jax.ShapeDtypeStruct((B, 3, N), jnp.int32),
            jax.ShapeDtypeStruct((B, 3, N), jnp.float32),
        ],
    )(xyz1t, xyz2)


def _sc_interp(idx_flat, w_flat, table, B, N, C2):
    BN = B * N
    info = plsc.get_sparse_core_info()
    NW = info.num_cores * info.num_subcores
    P = BN // NW
    CP = SC_CP
    NCH = P // CP
    WPB = N // P
    mesh = plsc.VectorSubcoreMesh(core_axis_name="c", subcore_axis_name="s")

    @functools.partial(
        pl.kernel,
        mesh=mesh,
        out_type=jax.ShapeDtypeStruct((BN, C2), jnp.float32),
        scratch_types=[
            pltpu.VMEM((CP,), jnp.int32),
            pltpu.VMEM((CP,), jnp.int32),
            pltpu.VMEM((CP,), jnp.int32),
            pltpu.VMEM((CP,), jnp.float32),
            pltpu.VMEM((CP,), jnp.float32),
            pltpu.VMEM((CP,), jnp.float32),
            pltpu.VMEM((CP, C2), jnp.float32),
            pltpu.VMEM((CP, C2), jnp.float32),
            pltpu.VMEM((CP, C2), jnp.float32),
            pltpu.SemaphoreType.DMA,
        ],
    )
    def k(idx_hbm, w_hbm, table_hbm, out_hbm,
          i0, i1, i2, w0, w1, w2, r0, r1, r2, sem):
        wid = lax.axis_index("s") * info.num_cores + lax.axis_index("c")
        base = wid * P
        b = wid // WPB
        n0 = (wid % WPB) * P
        idxv = [i0, i1, i2]
        wv = [w0, w1, w2]
        rv = [r0, r1, r2]

        def chunk(ci, _):
            nbase = n0 + ci * CP
            for kk in range(3):
                off = (b * 3 + kk) * N + nbase
                pltpu.sync_copy(idx_hbm.at[pl.ds(off, CP)], idxv[kk])
                pltpu.sync_copy(w_hbm.at[pl.ds(off, CP)], wv[kk])
            cps = [pltpu.async_copy(table_hbm.at[idxv[kk]], rv[kk], sem)
                   for kk in range(3)]
            for cp in cps:
                cp.wait()

            def group(g, _):
                wa = [wv[kk][pl.ds(g * 16, 16)] for kk in range(3)]
                for l in range(16):
                    a0 = wa[0][l]
                    a1 = wa[1][l]
                    a2 = wa[2][l]
                    p = g * 16 + l
                    for j in range(C2 // 16):
                        sl = pl.ds(j * 16, 16)
                        r0[p, sl] = (a0 * r0[p, sl] + a1 * r1[p, sl]
                                     + a2 * r2[p, sl])
                return 0

            lax.fori_loop(0, CP // 16, group, 0)
            pltpu.sync_copy(r0, out_hbm.at[pl.ds(base + ci * CP, CP)])
            return 0

        lax.fori_loop(0, NCH, chunk, 0)

    return k(idx_flat, w_flat, table)


def _mm1_body(C1, p1_ref, it_ref, w1t_ref, h1_ref, st_ref):
    @pl.when(pl.program_id(0) == 0)
    def _init():
        st_ref[...] = jnp.zeros_like(st_ref)

    w = w1t_ref[...]
    h = (jnp.dot(p1_ref[...], w[:C1], preferred_element_type=jnp.float32) +
         jnp.dot(it_ref[...], w[C1:], preferred_element_type=jnp.float32))
    h1_ref[...] = h
    st_ref[0:1, :] += jnp.sum(h, axis=0, keepdims=True)
    st_ref[1:2, :] += jnp.sum(h * h, axis=0, keepdims=True)


def _mm1(p1, interp, w1t):
    BN, C1 = p1.shape
    IN_CH, O1 = w1t.shape
    grid = (BN // MB,)
    return pl.pallas_call(
        functools.partial(_mm1_body, C1),
        grid=grid,
        in_specs=[
            pl.BlockSpec((MB, C1), lambda i: (i, 0)),
            pl.BlockSpec((MB, IN_CH - C1), lambda i: (i, 0)),
            pl.BlockSpec((IN_CH, O1), lambda i: (0, 0)),
        ],
        out_specs=[
            pl.BlockSpec((MB, O1), lambda i: (i, 0)),
            pl.BlockSpec((8, O1), lambda i: (0, 0)),
        ],
        out_shape=[
            jax.ShapeDtypeStruct((BN, O1), jnp.float32),
            jax.ShapeDtypeStruct((8, O1), jnp.float32),
        ],
    )(p1, interp, w1t)


def _mm2_body(M, h1_ref, st1_ref, g_ref, b_ref, w2t_ref, h2_ref, st2_ref):
    @pl.when(pl.program_id(0) == 0)
    def _init():
        st2_ref[...] = jnp.zeros_like(st2_ref)

    st = st1_ref[...]
    mean = st[0:1] * (1.0 / M)
    var = st[1:2] * (1.0 / M) - mean * mean
    scale = g_ref[...] * lax.rsqrt(var + EPS)
    shift = b_ref[...] - mean * scale
    h = jnp.maximum(h1_ref[...] * scale + shift, 0.0)
    h2 = jnp.dot(h, w2t_ref[...], preferred_element_type=jnp.float32)
    h2_ref[...] = h2
    st2_ref[0:1, :] += jnp.sum(h2, axis=0, keepdims=True)
    st2_ref[1:2, :] += jnp.sum(h2 * h2, axis=0, keepdims=True)


def _mm2(h1, st1, g, b, w2t):
    BN, O1 = h1.shape
    O2 = w2t.shape[1]
    grid = (BN // MB,)
    return pl.pallas_call(
        functools.partial(_mm2_body, BN),
        grid=grid,
        in_specs=[
            pl.BlockSpec((MB, O1), lambda i: (i, 0)),
            pl.BlockSpec((8, O1), lambda i: (0, 0)),
            pl.BlockSpec((1, O1), lambda i: (0, 0)),
            pl.BlockSpec((1, O1), lambda i: (0, 0)),
            pl.BlockSpec((O1, O2), lambda i: (0, 0)),
        ],
        out_specs=[
            pl.BlockSpec((MB, O2), lambda i: (i, 0)),
            pl.BlockSpec((8, O2), lambda i: (0, 0)),
        ],
        out_shape=[
            jax.ShapeDtypeStruct((BN, O2), jnp.float32),
            jax.ShapeDtypeStruct((8, O2), jnp.float32),
        ],
    )(h1, st1, g, b, w2t)


def _bn_relu_body(M, h_ref, st_ref, g_ref, b_ref, o_ref):
    st = st_ref[...]
    mean = st[0:1] * (1.0 / M)
    var = st[1:2] * (1.0 / M) - mean * mean
    scale = g_ref[...] * lax.rsqrt(var + EPS)
    shift = b_ref[...] - mean * scale
    o_ref[...] = jnp.maximum(h_ref[...] * scale + shift, 0.0)


def _bn_relu(h, st, g, b):
    BN, O = h.shape
    grid = (BN // MB,)
    return pl.pallas_call(
        functools.partial(_bn_relu_body, BN),
        grid=grid,
        in_specs=[
            pl.BlockSpec((MB, O), lambda i: (i, 0)),
            pl.BlockSpec((8, O), lambda i: (0, 0)),
            pl.BlockSpec((1, O), lambda i: (0, 0)),
            pl.BlockSpec((1, O), lambda i: (0, 0)),
        ],
        out_specs=pl.BlockSpec((MB, O), lambda i: (i, 0)),
        out_shape=jax.ShapeDtypeStruct((BN, O), jnp.float32),
    )(h, st, g, b)


def kernel(xyz1, xyz2, points1, points2, W1, g1, b1, W2, g2, b2):
    B, N, _ = xyz1.shape
    S = xyz2.shape[1]
    C1 = points1.shape[2]
    C2 = points2.shape[2]
    O1 = W1.shape[0]
    O2 = W2.shape[0]

    xyz1t = jnp.transpose(xyz1, (0, 2, 1))
    idx, w = _three_nn(xyz1t, xyz2)
    interp = _sc_interp(idx.reshape(-1), w.reshape(-1),
                        points2.reshape(B * S, C2), B, N, C2)
    h1, st1 = _mm1(points1.reshape(B * N, C1), interp, jnp.transpose(W1))
    h2, st2 = _mm2(h1, st1, g1.reshape(1, O1), b1.reshape(1, O1),
                   jnp.transpose(W2))
    out = _bn_relu(h2, st2, g2.reshape(1, O2), b2.reshape(1, O2))
    return out.reshape(B, N, O2)


# speedup vs baseline: 1.2522x; 1.2522x over previous
import functools

import jax
import jax.numpy as jnp
from jax import lax
from jax.experimental import pallas as pl
from jax.experimental.pallas import tpu as pltpu
from jax.experimental.pallas import tpu_sc as plsc

EPS = 1e-5
NBLK = 512
MB = 512
SC_CP = 128


def _three_nn_body(S, xyz1t_ref, xyz2_ref, idx_ref, w_ref):
    b = pl.program_id(0)
    x1t = xyz1t_ref[0]
    x2 = xyz2_ref[0]
    d = None
    for c in range(3):
        diff = x2[:, c:c + 1] - x1t[c:c + 1, :]
        d = diff * diff if d is None else d + diff * diff
    iota = lax.broadcasted_iota(jnp.int32, d.shape, 0)
    idxs, vals = [], []
    for k in range(3):
        m = jnp.min(d, axis=0, keepdims=True)
        i = jnp.min(jnp.where(d == m, iota, S), axis=0, keepdims=True)
        vals.append(m)
        idxs.append(i)
        if k < 2:
            d = jnp.where(iota == i, jnp.float32(1e30), d)
    recip = [1.0 / (v + 1e-8) for v in vals]
    norm = recip[0] + recip[1] + recip[2]
    idx_ref[0] = jnp.concatenate(idxs, axis=0) + b * S
    w_ref[0] = jnp.concatenate([r / norm for r in recip], axis=0)


def _three_nn(xyz1t, xyz2):
    B, _, N = xyz1t.shape
    S = xyz2.shape[1]
    grid = (B, N // NBLK)
    return pl.pallas_call(
        functools.partial(_three_nn_body, S),
        grid=grid,
        in_specs=[
            pl.BlockSpec((1, 3, NBLK), lambda b, i: (b, 0, i)),
            pl.BlockSpec((1, S, 3), lambda b, i: (b, 0, 0)),
        ],
        out_specs=[
            pl.BlockSpec((1, 3, NBLK), lambda b, i: (b, 0, i)),
            pl.BlockSpec((1, 3, NBLK), lambda b, i: (b, 0, i)),
        ],
        out_shape=[
            jax.ShapeDtypeStruct((B, 3, N), jnp.int32),
            jax.ShapeDtypeStruct((B, 3, N), jnp.float32),
        ],
    )(xyz1t, xyz2)


def _sc_interp(idx_flat, w_flat, table, B, N, C2):
    BN = B * N
    info = plsc.get_sparse_core_info()
    NW = info.num_cores * info.num_subcores
    P = BN // NW
    CP = SC_CP
    NCH = P // CP
    WPB = N // P
    mesh = plsc.VectorSubcoreMesh(core_axis_name="c", subcore_axis_name="s")

    @functools.partial(
        pl.kernel,
        mesh=mesh,
        out_type=jax.ShapeDtypeStruct((BN, C2), jnp.float32),
        scratch_types=[
            pltpu.VMEM((CP,), jnp.int32),
            pltpu.VMEM((CP,), jnp.int32),
            pltpu.VMEM((CP,), jnp.int32),
            pltpu.VMEM((CP,), jnp.float32),
            pltpu.VMEM((CP,), jnp.float32),
            pltpu.VMEM((CP,), jnp.float32),
            pltpu.VMEM((CP, C2), jnp.float32),
            pltpu.VMEM((CP, C2), jnp.float32),
            pltpu.VMEM((CP, C2), jnp.float32),
            pltpu.SemaphoreType.DMA,
        ],
    )
    def k(idx_hbm, w_hbm, table_hbm, out_hbm,
          i0, i1, i2, w0, w1, w2, r0, r1, r2, sem):
        wid = lax.axis_index("s") * info.num_cores + lax.axis_index("c")
        base = wid * P
        b = wid // WPB
        n0 = (wid % WPB) * P
        idxv = [i0, i1, i2]
        wv = [w0, w1, w2]
        rv = [r0, r1, r2]

        def chunk(ci, _):
            nbase = n0 + ci * CP
            for kk in range(3):
                off = (b * 3 + kk) * N + nbase
                pltpu.sync_copy(idx_hbm.at[pl.ds(off, CP)], idxv[kk])
                pltpu.sync_copy(w_hbm.at[pl.ds(off, CP)], wv[kk])
            cps = [pltpu.async_copy(table_hbm.at[idxv[kk]], rv[kk], sem)
                   for kk in range(3)]
            for cp in cps:
                cp.wait()

            def group(g, _):
                wa = [wv[kk][pl.ds(g * 16, 16)] for kk in range(3)]
                for l in range(16):
                    a0 = wa[0][l]
                    a1 = wa[1][l]
                    a2 = wa[2][l]
                    p = g * 16 + l
                    for j in range(C2 // 16):
                        sl = pl.ds(j * 16, 16)
                        r0[p, sl] = (a0 * r0[p, sl] + a1 * r1[p, sl]
                                     + a2 * r2[p, sl])
                return 0

            lax.fori_loop(0, CP // 16, group, 0)
            pltpu.sync_copy(r0, out_hbm.at[pl.ds(base + ci * CP, CP)])
            return 0

        lax.fori_loop(0, NCH, chunk, 0)

    return k(idx_flat, w_flat, table)


def _mm1_body(C1, p1_ref, it_ref, w1t_ref, h1_ref, st_ref):
    @pl.when(pl.program_id(0) == 0)
    def _init():
        st_ref[...] = jnp.zeros_like(st_ref)

    w = w1t_ref[...]
    h = (jnp.dot(p1_ref[...], w[:C1], preferred_element_type=jnp.float32) +
         jnp.dot(it_ref[...], w[C1:], preferred_element_type=jnp.float32))
    h1_ref[...] = h
    st_ref[0:1, :] += jnp.sum(h, axis=0, keepdims=True)
    st_ref[1:2, :] += jnp.sum(h * h, axis=0, keepdims=True)


def _mm1(p1, interp, w1t):
    BN, C1 = p1.shape
    IN_CH, O1 = w1t.shape
    grid = (BN // MB,)
    return pl.pallas_call(
        functools.partial(_mm1_body, C1),
        grid=grid,
        in_specs=[
            pl.BlockSpec((MB, C1), lambda i: (i, 0)),
            pl.BlockSpec((MB, IN_CH - C1), lambda i: (i, 0)),
            pl.BlockSpec((IN_CH, O1), lambda i: (0, 0)),
        ],
        out_specs=[
            pl.BlockSpec((MB, O1), lambda i: (i, 0)),
            pl.BlockSpec((8, O1), lambda i: (0, 0)),
        ],
        out_shape=[
            jax.ShapeDtypeStruct((BN, O1), jnp.float32),
            jax.ShapeDtypeStruct((8, O1), jnp.float32),
        ],
    )(p1, interp, w1t)


def _mm2_body(M, h1_ref, st1_ref, g_ref, b_ref, w2t_ref, h2_ref, st2_ref):
    @pl.when(pl.program_id(0) == 0)
    def _init():
        st2_ref[...] = jnp.zeros_like(st2_ref)

    st = st1_ref[...]
    mean = st[0:1] * (1.0 / M)
    var = st[1:2] * (1.0 / M) - mean * mean
    scale = g_ref[...] * lax.rsqrt(var + EPS)
    shift = b_ref[...] - mean * scale
    h = jnp.maximum(h1_ref[...] * scale + shift, 0.0)
    h2 = jnp.dot(h, w2t_ref[...], preferred_element_type=jnp.float32)
    h2_ref[...] = h2
    st2_ref[0:1, :] += jnp.sum(h2, axis=0, keepdims=True)
    st2_ref[1:2, :] += jnp.sum(h2 * h2, axis=0, keepdims=True)


def _mm2(h1, st1, g, b, w2t):
    BN, O1 = h1.shape
    O2 = w2t.shape[1]
    grid = (BN // MB,)
    return pl.pallas_call(
        functools.partial(_mm2_body, BN),
        grid=grid,
        in_specs=[
            pl.BlockSpec((MB, O1), lambda i: (i, 0)),
            pl.BlockSpec((8, O1), lambda i: (0, 0)),
            pl.BlockSpec((1, O1), lambda i: (0, 0)),
            pl.BlockSpec((1, O1), lambda i: (0, 0)),
            pl.BlockSpec((O1, O2), lambda i: (0, 0)),
        ],
        out_specs=[
            pl.BlockSpec((MB, O2), lambda i: (i, 0)),
            pl.BlockSpec((8, O2), lambda i: (0, 0)),
        ],
        out_shape=[
            jax.ShapeDtypeStruct((BN, O2), jnp.float32),
            jax.ShapeDtypeStruct((8, O2), jnp.float32),
        ],
    )(h1, st1, g, b, w2t)


def _bn_relu_body(M, h_ref, st_ref, g_ref, b_ref, o_ref):
    st = st_ref[...]
    mean = st[0:1] * (1.0 / M)
    var = st[1:2] * (1.0 / M) - mean * mean
    scale = g_ref[...] * lax.rsqrt(var + EPS)
    shift = b_ref[...] - mean * scale
    o_ref[...] = jnp.maximum(h_ref[...] * scale + shift, 0.0)


def _bn_relu(h, st, g, b):
    BN, O = h.shape
    grid = (BN // MB,)
    return pl.pallas_call(
        functools.partial(_bn_relu_body, BN),
        grid=grid,
        in_specs=[
            pl.BlockSpec((MB, O), lambda i: (i, 0)),
            pl.BlockSpec((8, O), lambda i: (0, 0)),
            pl.BlockSpec((1, O), lambda i: (0, 0)),
            pl.BlockSpec((1, O), lambda i: (0, 0)),
        ],
        out_specs=pl.BlockSpec((MB, O), lambda i: (i, 0)),
        out_shape=jax.ShapeDtypeStruct((BN, O), jnp.float32),
    )(h, st, g, b)


def kernel(xyz1, xyz2, points1, points2, W1, g1, b1, W2, g2, b2):
    B, N, _ = xyz1.shape
    S = xyz2.shape[1]
    C1 = points1.shape[2]
    C2 = points2.shape[2]
    O1 = W1.shape[0]
    O2 = W2.shape[0]

    xyz1t = jnp.transpose(xyz1, (0, 2, 1))
    idx, w = _three_nn(xyz1t, xyz2)
    interp = _sc_interp(idx.reshape(-1), w.reshape(-1),
                        points2.reshape(B * S, C2), B, N, C2)
    h1, st1 = _mm1(points1.reshape(B * N, C1), interp, jnp.transpose(W1))
    h2, st2 = _mm2(h1, st1, g1.reshape(1, O1), b1.reshape(1, O1),
                   jnp.transpose(W2))
    out = _bn_relu(h2, st2, g2.reshape(1, O2), b2.reshape(1, O2))
    return out.reshape(B, N, O2)


# speedup vs baseline: 1.3433x; 1.0727x over previous
import functools

import jax
import jax.numpy as jnp
from jax import lax
from jax.experimental import pallas as pl
from jax.experimental.pallas import tpu as pltpu
from jax.experimental.pallas import tpu_sc as plsc

EPS = 1e-5
NBLK = 512
MB = 512
SC_CP = 32


def _three_nn_body(S, xyz1t_ref, xyz2_ref, idx_ref, w_ref):
    b = pl.program_id(0)
    x1t = xyz1t_ref[0]
    x2 = xyz2_ref[0]
    d = None
    for c in range(3):
        diff = x2[:, c:c + 1] - x1t[c:c + 1, :]
        d = diff * diff if d is None else d + diff * diff
    iota = lax.broadcasted_iota(jnp.int32, d.shape, 0)
    idxs, vals = [], []
    for k in range(3):
        m = jnp.min(d, axis=0, keepdims=True)
        i = jnp.min(jnp.where(d == m, iota, S), axis=0, keepdims=True)
        vals.append(m)
        idxs.append(i)
        if k < 2:
            d = jnp.where(iota == i, jnp.float32(1e30), d)
    recip = [1.0 / (v + 1e-8) for v in vals]
    norm = recip[0] + recip[1] + recip[2]
    idx_ref[0] = jnp.concatenate(idxs, axis=0) + b * S
    w_ref[0] = jnp.concatenate([r / norm for r in recip], axis=0)


def _three_nn(xyz1t, xyz2):
    B, _, N = xyz1t.shape
    S = xyz2.shape[1]
    grid = (B, N // NBLK)
    return pl.pallas_call(
        functools.partial(_three_nn_body, S),
        grid=grid,
        in_specs=[
            pl.BlockSpec((1, 3, NBLK), lambda b, i: (b, 0, i)),
            pl.BlockSpec((1, S, 3), lambda b, i: (b, 0, 0)),
        ],
        out_specs=[
            pl.BlockSpec((1, 3, NBLK), lambda b, i: (b, 0, i)),
            pl.BlockSpec((1, 3, NBLK), lambda b, i: (b, 0, i)),
        ],
        out_shape=[
            jax.ShapeDtypeStruct((B, 3, N), jnp.int32),
            jax.ShapeDtypeStruct((B, 3, N), jnp.float32),
        ],
    )(xyz1t, xyz2)


def _sc_interp(idx_flat, w_flat, table, B, N, C2):
    BN = B * N
    info = plsc.get_sparse_core_info()
    NW = info.num_cores * info.num_subcores
    P = BN // NW
    CP = SC_CP
    NCH = P // CP
    WPB = N // P
    mesh = plsc.VectorSubcoreMesh(core_axis_name="c", subcore_axis_name="s")

    @functools.partial(
        pl.kernel,
        mesh=mesh,
        out_type=jax.ShapeDtypeStruct((BN, C2), jnp.float32),
        scratch_types=[
            pltpu.VMEM((NCH, 1, CP), jnp.int32),
            pltpu.VMEM((NCH, 1, CP), jnp.int32),
            pltpu.VMEM((NCH, 1, CP), jnp.int32),
            pltpu.VMEM((P,), jnp.float32),
            pltpu.VMEM((P,), jnp.float32),
            pltpu.VMEM((P,), jnp.float32),
            pltpu.VMEM((CP, C2), jnp.float32),
            pltpu.VMEM((CP, C2), jnp.float32),
            pltpu.VMEM((CP, C2), jnp.float32),
            pltpu.VMEM((CP, C2), jnp.float32),
            pltpu.VMEM((CP, C2), jnp.float32),
            pltpu.VMEM((CP, C2), jnp.float32),
            pltpu.VMEM((CP, C2), jnp.float32),
            pltpu.VMEM((CP, C2), jnp.float32),
            pltpu.SemaphoreType.DMA,
            pltpu.SemaphoreType.DMA,
            pltpu.SemaphoreType.DMA,
            pltpu.SemaphoreType.DMA,
        ],
    )
    def k(idx_hbm, w_hbm, table_hbm, out_hbm,
          i0, i1, i2, w0, w1, w2,
          r00, r01, r02, r10, r11, r12, ob0, ob1,
          gsem0, gsem1, wsem0, wsem1):
        wid = lax.axis_index("s") * info.num_cores + lax.axis_index("c")
        base = pl.multiple_of(wid * P, 8)
        b = wid // WPB
        n0 = (wid % WPB) * P
        iv = [i0, i1, i2]
        wv = [w0, w1, w2]
        rsets = [(r00, r01, r02), (r10, r11, r12)]
        obs = [ob0, ob1]
        gsems = [gsem0, gsem1]
        wsems = [wsem0, wsem1]

        for kk in range(3):
            off = (b * 3 + kk) * N + n0
            pltpu.sync_copy(
                idx_hbm.at[pl.ds(pl.multiple_of(off // CP, 8), NCH)], iv[kk])
            pltpu.sync_copy(w_hbm.at[pl.ds(pl.multiple_of(off, 8), P)], wv[kk])

        def fire(ci, s):
            for kk in range(3):
                pltpu.async_copy(
                    table_hbm.at[iv[kk].at[ci, 0]],
                    rsets[s][kk], gsems[s])

        def drain_gather(ci, s):
            for kk in range(3):
                pltpu.make_async_copy(
                    table_hbm.at[iv[kk].at[ci, 0]],
                    rsets[s][kk], gsems[s]).wait()


        def compute(ci, s):
            rv = rsets[s]
            ob = obs[s]

            def group(g, _):
                wbase = ci * CP + g * 16
                wa = [wv[kk][pl.ds(wbase, 16)] for kk in range(3)]
                for l in range(16):
                    a0 = wa[0][l]
                    a1 = wa[1][l]
                    a2 = wa[2][l]
                    p = g * 16 + l
                    for j in range(C2 // 16):
                        sl = pl.ds(j * 16, 16)
                        ob[p, sl] = (a0 * rv[0][p, sl] + a1 * rv[1][p, sl]
                                     + a2 * rv[2][p, sl])
                return 0

            lax.fori_loop(0, CP // 16, group, 0)

        def half(ci, s):
            drain_gather(ci, s)

            @pl.when(ci >= 2)
            def _():
                pltpu.make_async_copy(
                    obs[s], out_hbm.at[pl.ds(base, CP)], wsems[s]).wait()

            compute(ci, s)
            pltpu.async_copy(
                obs[s],
                out_hbm.at[pl.ds(pl.multiple_of(base + ci * CP, 8), CP)],
                wsems[s])

            @pl.when(ci + 2 < NCH)
            def _():
                fire(ci + 2, s)

        fire(0, 0)
        fire(1, 1)

        def pair(q, _):
            half(2 * q, 0)
            half(2 * q + 1, 1)
            return 0

        lax.fori_loop(0, NCH // 2, pair, 0)
        pltpu.make_async_copy(ob0, out_hbm.at[pl.ds(base, CP)], wsem0).wait()
        pltpu.make_async_copy(ob1, out_hbm.at[pl.ds(base, CP)], wsem1).wait()

    return k(idx_flat.reshape(-1, 1, CP), w_flat, table)


def _mlp_body(C1, M, p1_ref, ih_ref, w1t_ref, w2t_ref,
              g1_ref, b1_ref, g2_ref, b2_ref,
              o_ref, st1_ref, st2_ref, hbig_ref):
    ph = pl.program_id(0)
    i = pl.program_id(1)
    rows = pl.ds(i * MB, MB)

    @pl.when(ph == 0)
    def _phase0():
        @pl.when(i == 0)
        def _():
            st1_ref[...] = jnp.zeros_like(st1_ref)

        w = w1t_ref[...]
        h = (jnp.dot(p1_ref[...], w[:C1], preferred_element_type=jnp.float32)
             + jnp.dot(ih_ref[...], w[C1:], preferred_element_type=jnp.float32))
        hbig_ref[rows, :] = h
        st1_ref[0:1, :] += jnp.sum(h, axis=0, keepdims=True)
        st1_ref[1:2, :] += jnp.sum(h * h, axis=0, keepdims=True)

    @pl.when(ph == 1)
    def _phase1():
        @pl.when(i == 0)
        def _():
            st2_ref[...] = jnp.zeros_like(st2_ref)

        st = st1_ref[...]
        mean = st[0:1] * (1.0 / M)
        var = st[1:2] * (1.0 / M) - mean * mean
        scale = g1_ref[...] * lax.rsqrt(var + EPS)
        shift = b1_ref[...] - mean * scale
        h = jnp.maximum(hbig_ref[rows, :] * scale + shift, 0.0)
        h2 = jnp.dot(h, w2t_ref[...], preferred_element_type=jnp.float32)
        hbig_ref[rows, :] = h2
        st2_ref[0:1, :] += jnp.sum(h2, axis=0, keepdims=True)
        st2_ref[1:2, :] += jnp.sum(h2 * h2, axis=0, keepdims=True)

    @pl.when(ph == 2)
    def _phase2():
        st = st2_ref[...]
        mean = st[0:1] * (1.0 / M)
        var = st[1:2] * (1.0 / M) - mean * mean
        scale = g2_ref[...] * lax.rsqrt(var + EPS)
        shift = b2_ref[...] - mean * scale
        o_ref[...] = jnp.maximum(hbig_ref[rows, :] * scale + shift, 0.0)


def _mlp_fused(p1, interp, w1t, w2t, g1, b1, g2, b2):
    BN, C1 = p1.shape
    C2 = interp.shape[1]
    IN_CH, O1 = w1t.shape
    O2 = w2t.shape[1]
    assert O1 == O2, "in-place h scratch requires O1 == O2"
    nblk = BN // MB
    last = nblk - 1
    out, _, _ = pl.pallas_call(
        functools.partial(_mlp_body, C1, BN),
        grid=(3, nblk),
        in_specs=[
            pl.BlockSpec((MB, C1), lambda ph, i: (jnp.where(ph == 0, i, last), 0)),
            pl.BlockSpec((MB, C2), lambda ph, i: (jnp.where(ph == 0, i, last), 0)),
            pl.BlockSpec((IN_CH, O1), lambda ph, i: (0, 0)),
            pl.BlockSpec((O1, O2), lambda ph, i: (0, 0)),
            pl.BlockSpec((1, O1), lambda ph, i: (0, 0)),
            pl.BlockSpec((1, O1), lambda ph, i: (0, 0)),
            pl.BlockSpec((1, O2), lambda ph, i: (0, 0)),
            pl.BlockSpec((1, O2), lambda ph, i: (0, 0)),
        ],
        out_specs=[
            pl.BlockSpec((MB, O2), lambda ph, i: (jnp.where(ph == 2, i, 0), 0)),
            pl.BlockSpec((8, O1), lambda ph, i: (0, 0)),
            pl.BlockSpec((8, O2), lambda ph, i: (0, 0)),
        ],
        out_shape=[
            jax.ShapeDtypeStruct((BN, O2), jnp.float32),
            jax.ShapeDtypeStruct((8, O1), jnp.float32),
            jax.ShapeDtypeStruct((8, O2), jnp.float32),
        ],
        scratch_shapes=[pltpu.VMEM((BN, O1), jnp.float32)],
    )(p1, interp, w1t, w2t, g1, b1, g2, b2)
    return out


def kernel(xyz1, xyz2, points1, points2, W1, g1, b1, W2, g2, b2):
    B, N, _ = xyz1.shape
    S = xyz2.shape[1]
    C1 = points1.shape[2]
    C2 = points2.shape[2]
    O1 = W1.shape[0]
    O2 = W2.shape[0]

    xyz1t = jnp.transpose(xyz1, (0, 2, 1))
    idx, w = _three_nn(xyz1t, xyz2)
    interp = _sc_interp(idx.reshape(-1), w.reshape(-1),
                        points2.reshape(B * S, C2), B, N, C2)
    out = _mlp_fused(points1.reshape(B * N, C1), interp,
                     jnp.transpose(W1), jnp.transpose(W2),
                     g1.reshape(1, O1), b1.reshape(1, O1),
                     g2.reshape(1, O2), b2.reshape(1, O2))
    return out.reshape(B, N, O2)


# speedup vs baseline: 1.4550x; 1.0832x over previous
import functools

import jax
import jax.numpy as jnp
from jax import lax
from jax.experimental import pallas as pl
from jax.experimental.pallas import tpu as pltpu
from jax.experimental.pallas import tpu_sc as plsc

EPS = 1e-5
NBLK = 512
MB = 512
SC_CP = 64


def _three_nn_body(S, xyz1t_ref, xyz2_ref, idx_ref, w_ref):
    b = pl.program_id(0)
    x1t = xyz1t_ref[0]
    x2 = xyz2_ref[0]
    d = None
    for c in range(3):
        diff = x2[:, c:c + 1] - x1t[c:c + 1, :]
        d = diff * diff if d is None else d + diff * diff
    iota = lax.broadcasted_iota(jnp.int32, d.shape, 0)
    idxs, vals = [], []
    for k in range(3):
        m = jnp.min(d, axis=0, keepdims=True)
        i = jnp.min(jnp.where(d == m, iota, S), axis=0, keepdims=True)
        vals.append(m)
        idxs.append(i)
        if k < 2:
            d = jnp.where(iota == i, jnp.float32(1e30), d)
    recip = [1.0 / (v + 1e-8) for v in vals]
    norm = recip[0] + recip[1] + recip[2]
    idx_ref[0] = jnp.concatenate(idxs, axis=0) + b * S
    w_ref[0] = jnp.concatenate([r / norm for r in recip], axis=0)


def _three_nn(xyz1t, xyz2):
    B, _, N = xyz1t.shape
    S = xyz2.shape[1]
    grid = (B, N // NBLK)
    return pl.pallas_call(
        functools.partial(_three_nn_body, S),
        grid=grid,
        in_specs=[
            pl.BlockSpec((1, 3, NBLK), lambda b, i: (b, 0, i)),
            pl.BlockSpec((1, S, 3), lambda b, i: (b, 0, 0)),
        ],
        out_specs=[
            pl.BlockSpec((1, 3, NBLK), lambda b, i: (b, 0, i)),
            pl.BlockSpec((1, 3, NBLK), lambda b, i: (b, 0, i)),
        ],
        out_shape=[
            jax.ShapeDtypeStruct((B, 3, N), jnp.int32),
            jax.ShapeDtypeStruct((B, 3, N), jnp.float32),
        ],
    )(xyz1t, xyz2)


def _sc_interp(idx_flat, w_flat, table, B, N, C2):
    BN = B * N
    info = plsc.get_sparse_core_info()
    NW = info.num_cores * info.num_subcores
    P = BN // NW
    CP = SC_CP
    NCH = P // CP
    WPB = N // P
    mesh = plsc.VectorSubcoreMesh(core_axis_name="c", subcore_axis_name="s")

    @functools.partial(
        pl.kernel,
        mesh=mesh,
        out_type=jax.ShapeDtypeStruct((BN, C2), jnp.float32),
        scratch_types=[
            pltpu.VMEM((NCH, 1, CP), jnp.int32),
            pltpu.VMEM((NCH, 1, CP), jnp.int32),
            pltpu.VMEM((NCH, 1, CP), jnp.int32),
            pltpu.VMEM((P,), jnp.float32),
            pltpu.VMEM((P,), jnp.float32),
            pltpu.VMEM((P,), jnp.float32),
            pltpu.VMEM((CP, C2), jnp.float32),
            pltpu.VMEM((CP, C2), jnp.float32),
            pltpu.VMEM((CP, C2), jnp.float32),
            pltpu.VMEM((CP, C2), jnp.float32),
            pltpu.VMEM((CP, C2), jnp.float32),
            pltpu.VMEM((CP, C2), jnp.float32),
            pltpu.SemaphoreType.DMA,
            pltpu.SemaphoreType.DMA,
            pltpu.SemaphoreType.DMA,
        ],
    )
    def k(idx_hbm, w_hbm, table_hbm, out_hbm,
          i0, i1, i2, w0, w1, w2,
          r00, r01, r02, r10, r11, r12,
          gsem0, gsem1, wsem):
        wid = lax.axis_index("s") * info.num_cores + lax.axis_index("c")
        base = pl.multiple_of(wid * P, 8)
        b = wid // WPB
        n0 = (wid % WPB) * P
        iv = [i0, i1, i2]
        wv = [w0, w1, w2]
        rsets = [(r00, r01, r02), (r10, r11, r12)]
        gsems = [gsem0, gsem1]

        for kk in range(3):
            off = (b * 3 + kk) * N + n0
            pltpu.sync_copy(
                idx_hbm.at[pl.ds(pl.multiple_of(off // CP, 8), NCH)], iv[kk])
            pltpu.sync_copy(w_hbm.at[pl.ds(pl.multiple_of(off, 8), P)], wv[kk])

        def fire_k(ci, s, kk):
            pltpu.async_copy(
                table_hbm.at[iv[kk].at[ci, 0]],
                rsets[s][kk], gsems[s])

        def drain_gather(ci, s):
            for kk in range(3):
                pltpu.make_async_copy(
                    table_hbm.at[iv[kk].at[ci, 0]],
                    rsets[s][kk], gsems[s]).wait()

        def compute(ci, s):
            rv = rsets[s]

            def group(g, _):
                wbase = ci * CP + g * 16
                wa = [wv[kk][pl.ds(wbase, 16)] for kk in range(3)]
                for l in range(16):
                    a0 = wa[0][l]
                    a1 = wa[1][l]
                    a2 = wa[2][l]
                    p = g * 16 + l
                    for j in range(C2 // 16):
                        sl = pl.ds(j * 16, 16)
                        rv[0][p, sl] = (a0 * rv[0][p, sl] + a1 * rv[1][p, sl]
                                        + a2 * rv[2][p, sl])
                return 0

            lax.fori_loop(0, CP // 16, group, 0)

        def half(ci, s):
            other = 1 - s
            @pl.when(ci + 1 < NCH)
            def _():
                @pl.when(ci >= 1)
                def _():
                    pltpu.make_async_copy(
                        rsets[other][0], out_hbm.at[pl.ds(base, CP)],
                        wsem).wait()
                fire_k(ci + 1, other, 0)

            drain_gather(ci, s)
            compute(ci, s)
            pltpu.async_copy(
                rsets[s][0],
                out_hbm.at[pl.ds(pl.multiple_of(base + ci * CP, 8), CP)],
                wsem)

            @pl.when(ci + 2 < NCH)
            def _():
                fire_k(ci + 2, s, 1)
                fire_k(ci + 2, s, 2)

        for kk in range(3):
            fire_k(0, 0, kk)
        fire_k(1, 1, 1)
        fire_k(1, 1, 2)

        def pair(q, _):
            half(2 * q, 0)
            half(2 * q + 1, 1)
            return 0

        lax.fori_loop(0, NCH // 2, pair, 0)
        pltpu.make_async_copy(
            rsets[0][0], out_hbm.at[pl.ds(base, CP)], wsem).wait()
        pltpu.make_async_copy(
            rsets[1][0], out_hbm.at[pl.ds(base, CP)], wsem).wait()

    return k(idx_flat.reshape(-1, 1, CP), w_flat, table)


def _mlp_body(C1, M, p1_ref, ih_ref, w1t_ref, w2t_ref,
              g1_ref, b1_ref, g2_ref, b2_ref,
              o_ref, st1_ref, st2_ref, hbig_ref):
    ph = pl.program_id(0)
    i = pl.program_id(1)
    rows = pl.ds(i * MB, MB)

    @pl.when(ph == 0)
    def _phase0():
        @pl.when(i == 0)
        def _():
            st1_ref[...] = jnp.zeros_like(st1_ref)

        w = w1t_ref[...]
        h = (jnp.dot(p1_ref[...], w[:C1], preferred_element_type=jnp.float32)
             + jnp.dot(ih_ref[...], w[C1:], preferred_element_type=jnp.float32))
        hbig_ref[rows, :] = h
        st1_ref[0:1, :] += jnp.sum(h, axis=0, keepdims=True)
        st1_ref[1:2, :] += jnp.sum(h * h, axis=0, keepdims=True)

    @pl.when(ph == 1)
    def _phase1():
        @pl.when(i == 0)
        def _():
            st2_ref[...] = jnp.zeros_like(st2_ref)

        st = st1_ref[...]
        mean = st[0:1] * (1.0 / M)
        var = st[1:2] * (1.0 / M) - mean * mean
        scale = g1_ref[...] * lax.rsqrt(var + EPS)
        shift = b1_ref[...] - mean * scale
        h = jnp.maximum(hbig_ref[rows, :] * scale + shift, 0.0)
        h2 = jnp.dot(h, w2t_ref[...], preferred_element_type=jnp.float32)
        hbig_ref[rows, :] = h2
        st2_ref[0:1, :] += jnp.sum(h2, axis=0, keepdims=True)
        st2_ref[1:2, :] += jnp.sum(h2 * h2, axis=0, keepdims=True)

    @pl.when(ph == 2)
    def _phase2():
        st = st2_ref[...]
        mean = st[0:1] * (1.0 / M)
        var = st[1:2] * (1.0 / M) - mean * mean
        scale = g2_ref[...] * lax.rsqrt(var + EPS)
        shift = b2_ref[...] - mean * scale
        o_ref[...] = jnp.maximum(hbig_ref[rows, :] * scale + shift, 0.0)


def _mlp_fused(p1, interp, w1t, w2t, g1, b1, g2, b2):
    BN, C1 = p1.shape
    C2 = interp.shape[1]
    IN_CH, O1 = w1t.shape
    O2 = w2t.shape[1]
    assert O1 == O2, "in-place h scratch requires O1 == O2"
    nblk = BN // MB
    last = nblk - 1
    out, _, _ = pl.pallas_call(
        functools.partial(_mlp_body, C1, BN),
        grid=(3, nblk),
        in_specs=[
            pl.BlockSpec((MB, C1), lambda ph, i: (jnp.where(ph == 0, i, last), 0)),
            pl.BlockSpec((MB, C2), lambda ph, i: (jnp.where(ph == 0, i, last), 0)),
            pl.BlockSpec((IN_CH, O1), lambda ph, i: (0, 0)),
            pl.BlockSpec((O1, O2), lambda ph, i: (0, 0)),
            pl.BlockSpec((1, O1), lambda ph, i: (0, 0)),
            pl.BlockSpec((1, O1), lambda ph, i: (0, 0)),
            pl.BlockSpec((1, O2), lambda ph, i: (0, 0)),
            pl.BlockSpec((1, O2), lambda ph, i: (0, 0)),
        ],
        out_specs=[
            pl.BlockSpec((MB, O2), lambda ph, i: (jnp.where(ph == 2, i, 0), 0)),
            pl.BlockSpec((8, O1), lambda ph, i: (0, 0)),
            pl.BlockSpec((8, O2), lambda ph, i: (0, 0)),
        ],
        out_shape=[
            jax.ShapeDtypeStruct((BN, O2), jnp.float32),
            jax.ShapeDtypeStruct((8, O1), jnp.float32),
            jax.ShapeDtypeStruct((8, O2), jnp.float32),
        ],
        scratch_shapes=[pltpu.VMEM((BN, O1), jnp.float32)],
    )(p1, interp, w1t, w2t, g1, b1, g2, b2)
    return out


def kernel(xyz1, xyz2, points1, points2, W1, g1, b1, W2, g2, b2):
    B, N, _ = xyz1.shape
    S = xyz2.shape[1]
    C1 = points1.shape[2]
    C2 = points2.shape[2]
    O1 = W1.shape[0]
    O2 = W2.shape[0]

    xyz1t = jnp.transpose(xyz1, (0, 2, 1))
    idx, w = _three_nn(xyz1t, xyz2)
    interp = _sc_interp(idx.reshape(-1), w.reshape(-1),
                        points2.reshape(B * S, C2), B, N, C2)
    out = _mlp_fused(points1.reshape(B * N, C1), interp,
                     jnp.transpose(W1), jnp.transpose(W2),
                     g1.reshape(1, O1), b1.reshape(1, O1),
                     g2.reshape(1, O2), b2.reshape(1, O2))
    return out.reshape(B, N, O2)


# speedup vs baseline: 1.4810x; 1.0178x over previous
import functools

import jax
import jax.numpy as jnp
from jax import lax
from jax.experimental import pallas as pl
from jax.experimental.pallas import tpu as pltpu
from jax.experimental.pallas import tpu_sc as plsc

EPS = 1e-5
NBLK = 1024
MB = 512
SC_CP = 64


def _three_nn_body(S, xyz1t_ref, xyz2_ref, idx_ref, w_ref):
    b = pl.program_id(0)
    x1t = xyz1t_ref[0]
    x2 = xyz2_ref[0]
    d = None
    for c in range(3):
        diff = x2[:, c:c + 1] - x1t[c:c + 1, :]
        d = diff * diff if d is None else d + diff * diff
    iota = lax.broadcasted_iota(jnp.int32, d.shape, 0)
    idxs, vals = [], []
    for k in range(3):
        m = jnp.min(d, axis=0, keepdims=True)
        i = jnp.min(jnp.where(d == m, iota, S), axis=0, keepdims=True)
        vals.append(m)
        idxs.append(i)
        if k < 2:
            d = jnp.where(iota == i, jnp.float32(1e30), d)
    recip = [1.0 / (v + 1e-8) for v in vals]
    norm = recip[0] + recip[1] + recip[2]
    idx_ref[0] = jnp.concatenate(idxs, axis=0) + b * S
    w_ref[0] = jnp.concatenate([r / norm for r in recip], axis=0)


def _three_nn(xyz1t, xyz2):
    B, _, N = xyz1t.shape
    S = xyz2.shape[1]
    grid = (B, N // NBLK)
    return pl.pallas_call(
        functools.partial(_three_nn_body, S),
        grid=grid,
        in_specs=[
            pl.BlockSpec((1, 3, NBLK), lambda b, i: (b, 0, i)),
            pl.BlockSpec((1, S, 3), lambda b, i: (b, 0, 0)),
        ],
        out_specs=[
            pl.BlockSpec((1, 3, NBLK), lambda b, i: (b, 0, i)),
            pl.BlockSpec((1, 3, NBLK), lambda b, i: (b, 0, i)),
        ],
        out_shape=[
            jax.ShapeDtypeStruct((B, 3, N), jnp.int32),
            jax.ShapeDtypeStruct((B, 3, N), jnp.float32),
        ],
    )(xyz1t, xyz2)


def _sc_interp(idx_flat, w_flat, table, B, N, C2):
    BN = B * N
    info = plsc.get_sparse_core_info()
    NW = info.num_cores * info.num_subcores
    P = BN // NW
    CP = SC_CP
    NCH = P // CP
    WPB = N // P
    mesh = plsc.VectorSubcoreMesh(core_axis_name="c", subcore_axis_name="s")

    @functools.partial(
        pl.kernel,
        mesh=mesh,
        out_type=jax.ShapeDtypeStruct((BN, C2), jnp.float32),
        scratch_types=[
            pltpu.VMEM((NCH, 1, CP), jnp.int32),
            pltpu.VMEM((NCH, 1, CP), jnp.int32),
            pltpu.VMEM((NCH, 1, CP), jnp.int32),
            pltpu.VMEM((P,), jnp.float32),
            pltpu.VMEM((P,), jnp.float32),
            pltpu.VMEM((P,), jnp.float32),
            pltpu.VMEM((CP, C2), jnp.float32),
            pltpu.VMEM((CP, C2), jnp.float32),
            pltpu.VMEM((CP, C2), jnp.float32),
            pltpu.VMEM((CP, C2), jnp.float32),
            pltpu.VMEM((CP, C2), jnp.float32),
            pltpu.VMEM((CP, C2), jnp.float32),
            pltpu.SemaphoreType.DMA,
            pltpu.SemaphoreType.DMA,
            pltpu.SemaphoreType.DMA,
        ],
    )
    def k(idx_hbm, w_hbm, table_hbm, out_hbm,
          i0, i1, i2, w0, w1, w2,
          r00, r01, r02, r10, r11, r12,
          gsem0, gsem1, wsem):
        wid = lax.axis_index("s") * info.num_cores + lax.axis_index("c")
        base = pl.multiple_of(wid * P, 8)
        b = wid // WPB
        n0 = (wid % WPB) * P
        iv = [i0, i1, i2]
        wv = [w0, w1, w2]
        rsets = [(r00, r01, r02), (r10, r11, r12)]
        gsems = [gsem0, gsem1]

        for kk in range(3):
            off = (b * 3 + kk) * N + n0
            pltpu.sync_copy(
                idx_hbm.at[pl.ds(pl.multiple_of(off // CP, 8), NCH)], iv[kk])
            pltpu.sync_copy(w_hbm.at[pl.ds(pl.multiple_of(off, 8), P)], wv[kk])

        def fire_k(ci, s, kk):
            pltpu.async_copy(
                table_hbm.at[iv[kk].at[ci, 0]],
                rsets[s][kk], gsems[s])

        def drain_gather(ci, s):
            for kk in range(3):
                pltpu.make_async_copy(
                    table_hbm.at[iv[kk].at[ci, 0]],
                    rsets[s][kk], gsems[s]).wait()

        def compute(ci, s):
            rv = rsets[s]

            def group(g, _):
                wbase = ci * CP + g * 16
                wa = [wv[kk][pl.ds(wbase, 16)] for kk in range(3)]
                for l in range(16):
                    a0 = wa[0][l]
                    a1 = wa[1][l]
                    a2 = wa[2][l]
                    p = g * 16 + l
                    for j in range(C2 // 16):
                        sl = pl.ds(j * 16, 16)
                        rv[0][p, sl] = (a0 * rv[0][p, sl] + a1 * rv[1][p, sl]
                                        + a2 * rv[2][p, sl])
                return 0

            lax.fori_loop(0, CP // 16, group, 0)

        def half(ci, s):
            other = 1 - s
            @pl.when(ci + 1 < NCH)
            def _():
                @pl.when(ci >= 1)
                def _():
                    pltpu.make_async_copy(
                        rsets[other][0], out_hbm.at[pl.ds(base, CP)],
                        wsem).wait()
                fire_k(ci + 1, other, 0)

            drain_gather(ci, s)
            compute(ci, s)
            pltpu.async_copy(
                rsets[s][0],
                out_hbm.at[pl.ds(pl.multiple_of(base + ci * CP, 8), CP)],
                wsem)

            @pl.when(ci + 2 < NCH)
            def _():
                fire_k(ci + 2, s, 1)
                fire_k(ci + 2, s, 2)

        for kk in range(3):
            fire_k(0, 0, kk)
        fire_k(1, 1, 1)
        fire_k(1, 1, 2)

        def pair(q, _):
            half(2 * q, 0)
            half(2 * q + 1, 1)
            return 0

        lax.fori_loop(0, NCH // 2, pair, 0)
        pltpu.make_async_copy(
            rsets[0][0], out_hbm.at[pl.ds(base, CP)], wsem).wait()
        pltpu.make_async_copy(
            rsets[1][0], out_hbm.at[pl.ds(base, CP)], wsem).wait()

    return k(idx_flat.reshape(-1, 1, CP), w_flat, table)


def _mlp_body(C1, M, p1_ref, ih_ref, w1t_ref, w2t_ref,
              g1_ref, b1_ref, g2_ref, b2_ref,
              o_ref, st1_ref, st2_ref, hbig_ref):
    ph = pl.program_id(0)
    i = pl.program_id(1)
    rows = pl.ds(i * MB, MB)

    @pl.when(ph == 0)
    def _phase0():
        @pl.when(i == 0)
        def _():
            st1_ref[...] = jnp.zeros_like(st1_ref)

        w = w1t_ref[...]
        h = (jnp.dot(p1_ref[...], w[:C1], preferred_element_type=jnp.float32)
             + jnp.dot(ih_ref[...], w[C1:], preferred_element_type=jnp.float32))
        hbig_ref[rows, :] = h
        st1_ref[0:1, :] += jnp.sum(h, axis=0, keepdims=True)
        st1_ref[1:2, :] += jnp.sum(h * h, axis=0, keepdims=True)

    @pl.when(ph == 1)
    def _phase1():
        @pl.when(i == 0)
        def _():
            st2_ref[...] = jnp.zeros_like(st2_ref)

        st = st1_ref[...]
        mean = st[0:1] * (1.0 / M)
        var = st[1:2] * (1.0 / M) - mean * mean
        scale = g1_ref[...] * lax.rsqrt(var + EPS)
        shift = b1_ref[...] - mean * scale
        h = jnp.maximum(hbig_ref[rows, :] * scale + shift, 0.0)
        h2 = jnp.dot(h, w2t_ref[...], preferred_element_type=jnp.float32)
        hbig_ref[rows, :] = h2
        st2_ref[0:1, :] += jnp.sum(h2, axis=0, keepdims=True)
        st2_ref[1:2, :] += jnp.sum(h2 * h2, axis=0, keepdims=True)

    @pl.when(ph == 2)
    def _phase2():
        st = st2_ref[...]
        mean = st[0:1] * (1.0 / M)
        var = st[1:2] * (1.0 / M) - mean * mean
        scale = g2_ref[...] * lax.rsqrt(var + EPS)
        shift = b2_ref[...] - mean * scale
        o_ref[...] = jnp.maximum(hbig_ref[rows, :] * scale + shift, 0.0)


def _mlp_fused(p1, interp, w1t, w2t, g1, b1, g2, b2):
    BN, C1 = p1.shape
    C2 = interp.shape[1]
    IN_CH, O1 = w1t.shape
    O2 = w2t.shape[1]
    assert O1 == O2, "in-place h scratch requires O1 == O2"
    nblk = BN // MB
    last = nblk - 1
    out, _, _ = pl.pallas_call(
        functools.partial(_mlp_body, C1, BN),
        grid=(3, nblk),
        in_specs=[
            pl.BlockSpec((MB, C1), lambda ph, i: (jnp.where(ph == 0, i, last), 0)),
            pl.BlockSpec((MB, C2), lambda ph, i: (jnp.where(ph == 0, i, last), 0)),
            pl.BlockSpec((IN_CH, O1), lambda ph, i: (0, 0)),
            pl.BlockSpec((O1, O2), lambda ph, i: (0, 0)),
            pl.BlockSpec((1, O1), lambda ph, i: (0, 0)),
            pl.BlockSpec((1, O1), lambda ph, i: (0, 0)),
            pl.BlockSpec((1, O2), lambda ph, i: (0, 0)),
            pl.BlockSpec((1, O2), lambda ph, i: (0, 0)),
        ],
        out_specs=[
            pl.BlockSpec((MB, O2), lambda ph, i: (jnp.where(ph == 2, i, 0), 0)),
            pl.BlockSpec((8, O1), lambda ph, i: (0, 0)),
            pl.BlockSpec((8, O2), lambda ph, i: (0, 0)),
        ],
        out_shape=[
            jax.ShapeDtypeStruct((BN, O2), jnp.float32),
            jax.ShapeDtypeStruct((8, O1), jnp.float32),
            jax.ShapeDtypeStruct((8, O2), jnp.float32),
        ],
        scratch_shapes=[pltpu.VMEM((BN, O1), jnp.float32)],
    )(p1, interp, w1t, w2t, g1, b1, g2, b2)
    return out


def kernel(xyz1, xyz2, points1, points2, W1, g1, b1, W2, g2, b2):
    B, N, _ = xyz1.shape
    S = xyz2.shape[1]
    C1 = points1.shape[2]
    C2 = points2.shape[2]
    O1 = W1.shape[0]
    O2 = W2.shape[0]

    xyz1t = jnp.transpose(xyz1, (0, 2, 1))
    idx, w = _three_nn(xyz1t, xyz2)
    interp = _sc_interp(idx.reshape(-1), w.reshape(-1),
                        points2.reshape(B * S, C2), B, N, C2)
    out = _mlp_fused(points1.reshape(B * N, C1), interp,
                     jnp.transpose(W1), jnp.transpose(W2),
                     g1.reshape(1, O1), b1.reshape(1, O1),
                     g2.reshape(1, O2), b2.reshape(1, O2))
    return out.reshape(B, N, O2)
